# baseline (speedup 1.0000x reference)
import functools

import jax
import jax.numpy as jnp
from jax import lax
from jax.experimental import pallas as pl
from jax.experimental.pallas import tpu as pltpu
from jax.experimental.pallas import tpu_sc as plsc

S = 50
V = 39
D = 32
L = 16
CHUNK = 800
NBUF = 2


@functools.lru_cache(maxsize=None)
def _build(n_tokens: int):
    mesh = plsc.VectorSubcoreMesh(core_axis_name="c", subcore_axis_name="s")
    nc, ns = mesh.num_cores, mesh.num_subcores
    nw = nc * ns
    assert n_tokens % (nw * CHUNK) == 0
    b_per_w = n_tokens // nw
    n_chunks = b_per_w // CHUNK

    def body(x_hbm, ft_hbm, out_hbm, idx_v, rows_v, ft_v,
             in_sems, out_sems):
        wid = lax.axis_index("s") * nc + lax.axis_index("c")
        base = wid * b_per_w
        iota = lax.iota(jnp.int32, L)
        iota_d = iota * D

        def start_in(i):
            slot = lax.rem(i, NBUF)
            pltpu.async_copy(
                x_hbm.at[pl.ds(base + i * CHUNK, CHUNK)],
                idx_v.at[slot], in_sems.at[slot])

        pltpu.sync_copy(ft_hbm, ft_v)
        for b in range(min(NBUF, n_chunks)):
            start_in(b)

        def chunk_step(i, _):
            slot = lax.rem(i, NBUF)
            off = base + i * CHUNK
            pltpu.make_async_copy(
                x_hbm.at[pl.ds(off, CHUNK)], idx_v.at[slot],
                in_sems.at[slot]).wait()
            @pl.when(i >= NBUF)
            def _():
                pltpu.make_async_copy(
                    rows_v.at[slot],
                    out_hbm.at[pl.ds((base + (i - NBUF) * CHUNK) * D,
                                     CHUNK * D)],
                    out_sems.at[slot]).wait()

            @plsc.parallel_loop(0, CHUNK // L, step=1, unroll=4)
            def _(g):
                x_vec = idx_v[slot, pl.ds(g * L, L)]
                pos = lax.rem(off + g * L + iota, S)
                a_vec = (x_vec + pos * V) * D
                dst = iota_d + g * (L * D)
                for d in range(D):
                    dv = jnp.bitwise_and(iota + d, D - 1)
                    val = plsc.load_gather(ft_v, [a_vec + dv])
                    plsc.store_scatter(rows_v.at[slot], [dst + dv], val)
            @pl.when(i + NBUF < n_chunks)
            def _():
                start_in(i + NBUF)
            pltpu.async_copy(rows_v.at[slot],
                             out_hbm.at[pl.ds(off * D, CHUNK * D)],
                             out_sems.at[slot])
            return ()

        lax.fori_loop(0, n_chunks, chunk_step, ())
        for b in range(min(NBUF, n_chunks)):
            i = n_chunks - min(NBUF, n_chunks) + b
            slot = lax.rem(jnp.int32(i), NBUF)
            pltpu.make_async_copy(
                rows_v.at[slot],
                out_hbm.at[pl.ds((base + i * CHUNK) * D, CHUNK * D)],
                out_sems.at[slot]).wait()

    run = pl.kernel(
        body,
        out_type=jax.ShapeDtypeStruct((n_tokens * D,), jnp.float32),
        mesh=mesh,
        scratch_types=[
            pltpu.VMEM((NBUF, CHUNK), jnp.int32),
            pltpu.VMEM((NBUF, CHUNK * D), jnp.float32),
            pltpu.VMEM((S * V * D,), jnp.float32),
            pltpu.SemaphoreType.DMA((NBUF,)),
            pltpu.SemaphoreType.DMA((NBUF,)),
        ],
        compiler_params=pltpu.CompilerParams(
            use_tc_tiling_on_sc=False, needs_layout_passes=False,
            disable_bounds_checks=True),
    )
    return run


def kernel(x, table, pe):
    b, s = x.shape
    ft = (pe[0][:, None, :] + table[None, :, :]).reshape(S * V * D)
    out = _build(b * s)(x.reshape(-1), ft)
    return out.reshape(b, s, D)


# speedup vs baseline: 1.3241x; 1.3241x over previous
import functools

import jax
import jax.numpy as jnp
from jax import lax
from jax.experimental import pallas as pl
from jax.experimental.pallas import tpu as pltpu
from jax.experimental.pallas import tpu_sc as plsc

S = 50
V = 39
D = 32
L = 16
CHUNK = 640
NBUF = 3


@functools.lru_cache(maxsize=None)
def _build(n_tokens: int):
    mesh = plsc.VectorSubcoreMesh(core_axis_name="c", subcore_axis_name="s")
    nc, ns = mesh.num_cores, mesh.num_subcores
    nw = nc * ns
    assert n_tokens % (nw * CHUNK) == 0
    b_per_w = n_tokens // nw
    n_chunks = b_per_w // CHUNK

    def body(x_hbm, ft_hbm, out_hbm, idx_v, rows_v, ft_v,
             in_sems, out_sems):
        wid = lax.axis_index("s") * nc + lax.axis_index("c")
        base = wid * b_per_w
        iota = lax.iota(jnp.int32, L)
        iota_d = iota * D

        def start_in(i):
            slot = lax.rem(i, NBUF)
            pltpu.async_copy(
                x_hbm.at[pl.ds(base + i * CHUNK, CHUNK)],
                idx_v.at[slot], in_sems.at[slot])

        pltpu.sync_copy(ft_hbm, ft_v)
        for b in range(min(NBUF, n_chunks)):
            start_in(b)

        def chunk_step(i, _):
            slot = lax.rem(i, NBUF)
            off = base + i * CHUNK
            pltpu.make_async_copy(
                x_hbm.at[pl.ds(off, CHUNK)], idx_v.at[slot],
                in_sems.at[slot]).wait()
            @pl.when(i >= NBUF)
            def _():
                pltpu.make_async_copy(
                    rows_v.at[slot],
                    out_hbm.at[pl.ds((base + (i - NBUF) * CHUNK) * D,
                                     CHUNK * D)],
                    out_sems.at[slot]).wait()

            @plsc.parallel_loop(0, CHUNK // L, step=1, unroll=2)
            def _(g):
                x_vec = idx_v[slot, pl.ds(g * L, L)]
                pos = lax.rem(off + g * L + iota, S)
                a_vec = (x_vec + pos * V) * D
                dst = iota_d + g * (L * D)
                for d in range(D):
                    dv = jnp.bitwise_and(iota + d, D - 1)
                    val = plsc.load_gather(ft_v, [a_vec + dv])
                    plsc.store_scatter(rows_v.at[slot], [dst + dv], val)
            @pl.when(i + NBUF < n_chunks)
            def _():
                start_in(i + NBUF)
            pltpu.async_copy(rows_v.at[slot],
                             out_hbm.at[pl.ds(off * D, CHUNK * D)],
                             out_sems.at[slot])
            return ()

        lax.fori_loop(0, n_chunks, chunk_step, ())
        for b in range(min(NBUF, n_chunks)):
            i = n_chunks - min(NBUF, n_chunks) + b
            slot = lax.rem(jnp.int32(i), NBUF)
            pltpu.make_async_copy(
                rows_v.at[slot],
                out_hbm.at[pl.ds((base + i * CHUNK) * D, CHUNK * D)],
                out_sems.at[slot]).wait()

    run = pl.kernel(
        body,
        out_type=jax.ShapeDtypeStruct((n_tokens * D,), jnp.float32),
        mesh=mesh,
        scratch_types=[
            pltpu.VMEM((NBUF, CHUNK), jnp.int32),
            pltpu.VMEM((NBUF, CHUNK * D), jnp.float32),
            pltpu.VMEM((S * V * D,), jnp.float32),
            pltpu.SemaphoreType.DMA((NBUF,)),
            pltpu.SemaphoreType.DMA((NBUF,)),
        ],
        compiler_params=pltpu.CompilerParams(
            use_tc_tiling_on_sc=False, needs_layout_passes=False,
            disable_bounds_checks=True),
    )
    return run


def kernel(x, table, pe):
    b, s = x.shape
    ft = (pe[0][:, None, :] + table[None, :, :]).reshape(S * V * D)
    out = _build(b * s)(x.reshape(-1), ft)
    return out.reshape(b, s, D)


# speedup vs baseline: 1.3251x; 1.0008x over previous
import functools

import jax
import jax.numpy as jnp
from jax import lax
from jax.experimental import pallas as pl
from jax.experimental.pallas import tpu as pltpu
from jax.experimental.pallas import tpu_sc as plsc

S = 50
V = 39
D = 32
L = 16
CHUNK = 800
NBUF = 2


@functools.lru_cache(maxsize=None)
def _build(n_tokens: int):
    mesh = plsc.VectorSubcoreMesh(core_axis_name="c", subcore_axis_name="s")
    nc, ns = mesh.num_cores, mesh.num_subcores
    nw = nc * ns
    assert n_tokens % (nw * CHUNK) == 0
    b_per_w = n_tokens // nw
    n_chunks = b_per_w // CHUNK

    def body(x_hbm, ft_hbm, out_hbm, idx_v, rows_v, ft_v,
             in_sems, out_sems):
        wid = lax.axis_index("s") * nc + lax.axis_index("c")
        base = wid * b_per_w
        iota = lax.iota(jnp.int32, L)
        iota_d = iota * D

        def start_in(i):
            slot = lax.rem(i, NBUF)
            pltpu.async_copy(
                x_hbm.at[pl.ds(base + i * CHUNK, CHUNK)],
                idx_v.at[slot], in_sems.at[slot])

        pltpu.sync_copy(ft_hbm, ft_v)
        for b in range(min(NBUF, n_chunks)):
            start_in(b)

        def chunk_step(i, _):
            slot = lax.rem(i, NBUF)
            off = base + i * CHUNK
            pltpu.make_async_copy(
                x_hbm.at[pl.ds(off, CHUNK)], idx_v.at[slot],
                in_sems.at[slot]).wait()
            @pl.when(i >= NBUF)
            def _():
                pltpu.make_async_copy(
                    rows_v.at[slot],
                    out_hbm.at[pl.ds((base + (i - NBUF) * CHUNK) * D,
                                     CHUNK * D)],
                    out_sems.at[slot]).wait()

            @plsc.parallel_loop(0, CHUNK // L, step=1, unroll=2)
            def _(g):
                x_vec = idx_v[slot, pl.ds(g * L, L)]
                pos = lax.rem(off + g * L + iota, S)
                a_vec = (x_vec + pos * V) * D
                dst = iota_d + g * (L * D)
                for d in range(D):
                    dv = jnp.bitwise_and(iota + d, D - 1)
                    val = plsc.load_gather(ft_v, [a_vec + dv])
                    plsc.store_scatter(rows_v.at[slot], [dst + dv], val)
            @pl.when(i + NBUF < n_chunks)
            def _():
                start_in(i + NBUF)
            pltpu.async_copy(rows_v.at[slot],
                             out_hbm.at[pl.ds(off * D, CHUNK * D)],
                             out_sems.at[slot])
            return ()

        lax.fori_loop(0, n_chunks, chunk_step, ())
        for b in range(min(NBUF, n_chunks)):
            i = n_chunks - min(NBUF, n_chunks) + b
            slot = lax.rem(jnp.int32(i), NBUF)
            pltpu.make_async_copy(
                rows_v.at[slot],
                out_hbm.at[pl.ds((base + i * CHUNK) * D, CHUNK * D)],
                out_sems.at[slot]).wait()

    run = pl.kernel(
        body,
        out_type=jax.ShapeDtypeStruct((n_tokens * D,), jnp.float32),
        mesh=mesh,
        scratch_types=[
            pltpu.VMEM((NBUF, CHUNK), jnp.int32),
            pltpu.VMEM((NBUF, CHUNK * D), jnp.float32),
            pltpu.VMEM((S * V * D,), jnp.float32),
            pltpu.SemaphoreType.DMA((NBUF,)),
            pltpu.SemaphoreType.DMA((NBUF,)),
        ],
        compiler_params=pltpu.CompilerParams(
            use_tc_tiling_on_sc=False, needs_layout_passes=False,
            disable_bounds_checks=True),
    )
    return run


def kernel(x, table, pe):
    b, s = x.shape
    ft = (pe[0][:, None, :] + table[None, :, :]).reshape(S * V * D)
    out = _build(b * s)(x.reshape(-1), ft)
    return out.reshape(b, s, D)
